# Initial kernel scaffold; baseline (speedup 1.0000x reference)
import jax
import jax.numpy as jnp
from jax.experimental import pallas as pl


def kernel(x, edge_index, batch, W1, b1, W2, b2):
    raise NotImplementedError("write your pallas kernel here")


# speedup vs baseline: 12.9389x; 12.9389x over previous
import jax
import jax.numpy as jnp
from jax import lax
from jax.experimental import pallas as pl
from jax.experimental.pallas import tpu as pltpu
from jax.experimental.pallas import tpu_sc as plsc

NC, NS, L = 2, 16, 16
NW = NC * NS

N = 10000
NPAD = 10240
D = 128
DH = D // NC
E = 320000
CB = 128
NCH = 160
EPT = NCH * CB
EPAD = EPT * NS
JUNK = N + 16
RPT = NPAD // NS

_mesh = plsc.VectorSubcoreMesh(
    core_axis_name="c", subcore_axis_name="s", num_cores=NC, num_subcores=NS)
_sc_params = pltpu.CompilerParams(
    needs_layout_passes=False, use_tc_tiling_on_sc=False)


def _deg_body(dst_hbm, out_hbm, dst_v, deg_v):
    cid = lax.axis_index("c")
    sid = lax.axis_index("s")
    wid = cid * NS + sid
    pltpu.sync_copy(dst_hbm.at[wid], dst_v)
    zeros = jnp.zeros((L,), jnp.float32)

    @pl.loop(0, NPAD // L)
    def _z(i):
        deg_v[pl.ds(i * L, L)] = zeros

    ones = jnp.ones((L,), jnp.float32)

    @pl.loop(0, (E // NW) // L)
    def _s(i):
        idx = dst_v[pl.ds(i * L, L)]
        plsc.addupdate_scatter(deg_v, [idx], ones)

    pltpu.sync_copy(deg_v, out_hbm.at[wid])


def _deg_call(dst):
    k = pl.kernel(
        _deg_body,
        out_type=jax.ShapeDtypeStruct((NW, NPAD), jnp.float32),
        mesh=_mesh,
        compiler_params=_sc_params,
        scratch_types=[
            pltpu.VMEM((E // NW,), jnp.int32),
            pltpu.VMEM((NPAD,), jnp.float32),
        ],
    )
    return k(dst.reshape(NW, E // NW))


def _agg_body(v_hbm, src_hbm, dst_hbm, z_hbm, out_hbm,
              src_v, dst_v, rows_v, acc_sh, sems):
    cid = lax.axis_index("c")
    sid = lax.axis_index("s")
    pltpu.sync_copy(z_hbm.at[pl.ds(sid * RPT, RPT)],
                    acc_sh.at[pl.ds(sid * RPT, RPT)])
    pltpu.sync_copy(src_hbm.at[sid], src_v)
    pltpu.sync_copy(dst_hbm.at[sid], dst_v)
    plsc.subcore_barrier()

    pltpu.async_copy(v_hbm.at[cid].at[src_v.at[0]], rows_v.at[0], sems.at[0])

    @pl.loop(0, NCH, step=2)
    def _batches(j0):
        for p in range(2):
            j = j0 + p
            pltpu.make_async_copy(
                v_hbm.at[cid].at[src_v.at[j]], rows_v.at[p], sems.at[p]).wait()

            @pl.when(j + 1 < NCH)
            def _prefetch():
                pltpu.async_copy(
                    v_hbm.at[cid].at[src_v.at[j + 1]], rows_v.at[1 - p],
                    sems.at[1 - p])

            pltpu.sync_copy(rows_v.at[p], acc_sh.at[dst_v.at[j]], add=True)

    plsc.subcore_barrier()
    pltpu.sync_copy(acc_sh.at[pl.ds(sid * RPT, RPT)],
                    out_hbm.at[cid, pl.ds(sid * RPT, RPT)])


def _agg_call(v2, srcp, dstp, zrows):
    k = pl.kernel(
        _agg_body,
        out_type=jax.ShapeDtypeStruct((NC, NPAD, DH), jnp.float32),
        mesh=_mesh,
        compiler_params=_sc_params,
        scratch_types=[
            pltpu.VMEM((NCH, CB), jnp.int32),
            pltpu.VMEM((NCH, CB), jnp.int32),
            pltpu.VMEM((2, CB, DH), jnp.float32),
            pltpu.VMEM_SHARED((NPAD, DH), jnp.float32),
            pltpu.SemaphoreType.DMA((2,)),
        ],
    )
    return k(v2, srcp, dstp, zrows)


def _dinv_body(p_ref, o_ref):
    s = jnp.sum(p_ref[...], axis=0, keepdims=True)
    o_ref[...] = lax.rsqrt(1.0 + s)


def _dinv_call(parts):
    return pl.pallas_call(
        _dinv_body,
        out_shape=jax.ShapeDtypeStruct((1, NPAD), jnp.float32),
    )(parts)


def _scale_body(x_ref, d_ref, o_ref):
    xv = x_ref[...] * d_ref[...]
    o_ref[0] = xv[:, :DH]
    o_ref[1] = xv[:, DH:]


def _scale_call(x, dcol):
    rb = 1000
    return pl.pallas_call(
        _scale_body,
        grid=(N // rb,),
        in_specs=[
            pl.BlockSpec((rb, D), lambda i: (i, 0)),
            pl.BlockSpec((rb, 1), lambda i: (i, 0)),
        ],
        out_specs=pl.BlockSpec((NC, rb, DH), lambda i: (0, i, 0)),
        out_shape=jax.ShapeDtypeStruct((NC, N, DH), jnp.float32),
    )(x, dcol)


def _mid_body(xs_ref, p_ref, d_ref, w1_ref, b1_ref, w2_ref, o_ref):
    d = d_ref[...]
    xv = jnp.concatenate([xs_ref[0], xs_ref[1]], axis=1)
    pv = jnp.concatenate([p_ref[0], p_ref[1]], axis=1)
    t = d * (xv + pv)
    h = jnp.dot(t, w1_ref[...], preferred_element_type=jnp.float32)
    h = jax.nn.gelu(h + b1_ref[...])
    y = jnp.dot(h, w2_ref[...], preferred_element_type=jnp.float32)
    yd = y * d
    o_ref[0] = yd[:, :DH]
    o_ref[1] = yd[:, DH:]


def _mid_call(xs2, p2, dcol, W1, b1, W2):
    rb = 1000
    dh = W1.shape[1]
    return pl.pallas_call(
        _mid_body,
        grid=(N // rb,),
        in_specs=[
            pl.BlockSpec((NC, rb, DH), lambda i: (0, i, 0)),
            pl.BlockSpec((NC, rb, DH), lambda i: (0, i, 0)),
            pl.BlockSpec((rb, 1), lambda i: (i, 0)),
            pl.BlockSpec((D, dh), lambda i: (0, 0)),
            pl.BlockSpec((1, dh), lambda i: (0, 0)),
            pl.BlockSpec((dh, D), lambda i: (0, 0)),
        ],
        out_specs=pl.BlockSpec((NC, rb, DH), lambda i: (0, i, 0)),
        out_shape=jax.ShapeDtypeStruct((NC, N, DH), jnp.float32),
    )(xs2, p2, dcol, W1, b1.reshape(1, dh), W2)


def _fin_body(ys_ref, q_ref, d_ref, b2_ref, o_ref):
    yv = jnp.concatenate([ys_ref[0], ys_ref[1]], axis=1)
    qv = jnp.concatenate([q_ref[0], q_ref[1]], axis=1)
    t = d_ref[...] * (yv + qv)
    o_ref[...] = jax.nn.gelu(t + b2_ref[...])


def _fin_call(ys2, q2, dcol, b2):
    rb = 1000
    return pl.pallas_call(
        _fin_body,
        grid=(N // rb,),
        in_specs=[
            pl.BlockSpec((NC, rb, DH), lambda i: (0, i, 0)),
            pl.BlockSpec((NC, rb, DH), lambda i: (0, i, 0)),
            pl.BlockSpec((rb, 1), lambda i: (i, 0)),
            pl.BlockSpec((1, D), lambda i: (0, 0)),
        ],
        out_specs=pl.BlockSpec((rb, D), lambda i: (i, 0)),
        out_shape=jax.ShapeDtypeStruct((N, D), jnp.float32),
    )(ys2, q2, dcol, b2.reshape(1, D))


def kernel(x, edge_index, batch, W1, b1, W2, b2):
    src = edge_index[0]
    dst = edge_index[1]
    pad = EPAD - E
    srcp = jnp.concatenate(
        [src, jnp.zeros((pad,), jnp.int32)]).reshape(NS, NCH, CB)
    dstp = jnp.concatenate(
        [dst, jnp.full((pad,), JUNK, jnp.int32)]).reshape(NS, NCH, CB)
    zrows = jnp.zeros((NPAD, DH), jnp.float32)
    srcp, dstp, zrows = lax.optimization_barrier((srcp, dstp, zrows))

    deg_parts = _deg_call(dst)
    dinv = _dinv_call(deg_parts)
    dcol = dinv.reshape(NPAD, 1)[:N]

    xs2 = _scale_call(x, dcol)
    p2 = _agg_call(xs2, srcp, dstp, zrows)
    ys2 = _mid_call(xs2, p2[:, :N], dcol, W1, b1, W2)
    q2 = _agg_call(ys2, srcp, dstp, zrows)
    out = _fin_call(ys2, q2[:, :N], dcol, b2)
    return (out, None)


# speedup vs baseline: 15.4715x; 1.1957x over previous
import jax
import jax.numpy as jnp
from jax import lax
from jax.experimental import pallas as pl
from jax.experimental.pallas import tpu as pltpu
from jax.experimental.pallas import tpu_sc as plsc

NC, NS, L = 2, 16, 16
NW = NC * NS

N = 10000
NPAD = 10240
D = 128
DH = D // NC
E = 320000
CB = 128
NCH = 160
EPT = NCH * CB
EPAD = EPT * NS
JUNK = N + 16
RPT = NPAD // NS

_mesh = plsc.VectorSubcoreMesh(
    core_axis_name="c", subcore_axis_name="s", num_cores=NC, num_subcores=NS)
_sc_params = pltpu.CompilerParams(
    needs_layout_passes=False, use_tc_tiling_on_sc=False)


def _deg_body(dst_hbm, out_hbm, dst_v, deg_v):
    cid = lax.axis_index("c")
    sid = lax.axis_index("s")
    wid = cid * NS + sid
    pltpu.sync_copy(dst_hbm.at[wid], dst_v)
    zeros = jnp.zeros((L,), jnp.float32)

    @pl.loop(0, NPAD // L)
    def _z(i):
        deg_v[pl.ds(i * L, L)] = zeros

    ones = jnp.ones((L,), jnp.float32)

    @pl.loop(0, (E // NW) // L)
    def _s(i):
        idx = dst_v[pl.ds(i * L, L)]
        plsc.addupdate_scatter(deg_v, [idx], ones)

    pltpu.sync_copy(deg_v, out_hbm.at[wid])


def _deg_call(dst):
    k = pl.kernel(
        _deg_body,
        out_type=jax.ShapeDtypeStruct((NW, NPAD), jnp.float32),
        mesh=_mesh,
        compiler_params=_sc_params,
        scratch_types=[
            pltpu.VMEM((E // NW,), jnp.int32),
            pltpu.VMEM((NPAD,), jnp.float32),
        ],
    )
    return k(dst.reshape(NW, E // NW))


NBUF = 4


def _agg_body(v_hbm, src_hbm, dst_hbm, z_hbm, out_hbm,
              src_v, dst_v, rows_v, acc_sh, gsems, ssems):
    cid = lax.axis_index("c")
    sid = lax.axis_index("s")
    pltpu.sync_copy(z_hbm.at[pl.ds(sid * RPT, RPT)],
                    acc_sh.at[pl.ds(sid * RPT, RPT)])
    pltpu.sync_copy(src_hbm.at[sid], src_v)
    pltpu.sync_copy(dst_hbm.at[sid], dst_v)
    plsc.subcore_barrier()

    for p in range(NBUF):
        pltpu.async_copy(v_hbm.at[cid].at[src_v.at[p]], rows_v.at[p],
                         gsems.at[p])

    @pl.loop(0, NCH, step=NBUF)
    def _batches(j0):
        for p in range(NBUF):
            j = j0 + p
            pltpu.make_async_copy(
                v_hbm.at[cid].at[src_v.at[j]], rows_v.at[p],
                gsems.at[p]).wait()
            pltpu.async_copy(rows_v.at[p], acc_sh.at[dst_v.at[j]],
                             ssems.at[p], add=True)

            @pl.when(j + NBUF < NCH)
            def _refill():
                pltpu.make_async_copy(
                    rows_v.at[p], acc_sh.at[dst_v.at[j]], ssems.at[p]).wait()
                pltpu.async_copy(v_hbm.at[cid].at[src_v.at[j + NBUF]],
                                 rows_v.at[p], gsems.at[p])

    for p in range(NBUF):
        j = NCH - NBUF + p
        pltpu.make_async_copy(
            rows_v.at[p], acc_sh.at[dst_v.at[j]], ssems.at[p]).wait()

    plsc.subcore_barrier()
    pltpu.sync_copy(acc_sh.at[pl.ds(sid * RPT, RPT)],
                    out_hbm.at[cid, pl.ds(sid * RPT, RPT)])


def _agg_call(v2, srcp, dstp, zrows):
    k = pl.kernel(
        _agg_body,
        out_type=jax.ShapeDtypeStruct((NC, NPAD, DH), jnp.float32),
        mesh=_mesh,
        compiler_params=_sc_params,
        scratch_types=[
            pltpu.VMEM((NCH, CB), jnp.int32),
            pltpu.VMEM((NCH, CB), jnp.int32),
            pltpu.VMEM((NBUF, CB, DH), jnp.float32),
            pltpu.VMEM_SHARED((NPAD, DH), jnp.float32),
            pltpu.SemaphoreType.DMA((NBUF,)),
            pltpu.SemaphoreType.DMA((NBUF,)),
        ],
    )
    return k(v2, srcp, dstp, zrows)


def _dinv_body(p_ref, o_ref):
    s = jnp.sum(p_ref[...], axis=0, keepdims=True)
    o_ref[...] = lax.rsqrt(1.0 + s)


def _dinv_call(parts):
    return pl.pallas_call(
        _dinv_body,
        out_shape=jax.ShapeDtypeStruct((1, NPAD), jnp.float32),
    )(parts)


def _scale_body(x_ref, d_ref, o_ref):
    xv = x_ref[...] * d_ref[...]
    o_ref[0] = xv[:, :DH]
    o_ref[1] = xv[:, DH:]


def _scale_call(x, dcol):
    rb = 1000
    return pl.pallas_call(
        _scale_body,
        grid=(N // rb,),
        in_specs=[
            pl.BlockSpec((rb, D), lambda i: (i, 0)),
            pl.BlockSpec((rb, 1), lambda i: (i, 0)),
        ],
        out_specs=pl.BlockSpec((NC, rb, DH), lambda i: (0, i, 0)),
        out_shape=jax.ShapeDtypeStruct((NC, N, DH), jnp.float32),
    )(x, dcol)


def _mid_body(xs_ref, p_ref, d_ref, w1_ref, b1_ref, w2_ref, o_ref):
    d = d_ref[...]
    xv = jnp.concatenate([xs_ref[0], xs_ref[1]], axis=1)
    pv = jnp.concatenate([p_ref[0], p_ref[1]], axis=1)
    t = d * (xv + pv)
    h = jnp.dot(t, w1_ref[...], preferred_element_type=jnp.float32)
    h = jax.nn.gelu(h + b1_ref[...])
    y = jnp.dot(h, w2_ref[...], preferred_element_type=jnp.float32)
    yd = y * d
    o_ref[0] = yd[:, :DH]
    o_ref[1] = yd[:, DH:]


def _mid_call(xs2, p2, dcol, W1, b1, W2):
    rb = 1000
    dh = W1.shape[1]
    return pl.pallas_call(
        _mid_body,
        grid=(N // rb,),
        in_specs=[
            pl.BlockSpec((NC, rb, DH), lambda i: (0, i, 0)),
            pl.BlockSpec((NC, rb, DH), lambda i: (0, i, 0)),
            pl.BlockSpec((rb, 1), lambda i: (i, 0)),
            pl.BlockSpec((D, dh), lambda i: (0, 0)),
            pl.BlockSpec((1, dh), lambda i: (0, 0)),
            pl.BlockSpec((dh, D), lambda i: (0, 0)),
        ],
        out_specs=pl.BlockSpec((NC, rb, DH), lambda i: (0, i, 0)),
        out_shape=jax.ShapeDtypeStruct((NC, N, DH), jnp.float32),
    )(xs2, p2, dcol, W1, b1.reshape(1, dh), W2)


def _fin_body(ys_ref, q_ref, d_ref, b2_ref, o_ref):
    yv = jnp.concatenate([ys_ref[0], ys_ref[1]], axis=1)
    qv = jnp.concatenate([q_ref[0], q_ref[1]], axis=1)
    t = d_ref[...] * (yv + qv)
    o_ref[...] = jax.nn.gelu(t + b2_ref[...])


def _fin_call(ys2, q2, dcol, b2):
    rb = 1000
    return pl.pallas_call(
        _fin_body,
        grid=(N // rb,),
        in_specs=[
            pl.BlockSpec((NC, rb, DH), lambda i: (0, i, 0)),
            pl.BlockSpec((NC, rb, DH), lambda i: (0, i, 0)),
            pl.BlockSpec((rb, 1), lambda i: (i, 0)),
            pl.BlockSpec((1, D), lambda i: (0, 0)),
        ],
        out_specs=pl.BlockSpec((rb, D), lambda i: (i, 0)),
        out_shape=jax.ShapeDtypeStruct((N, D), jnp.float32),
    )(ys2, q2, dcol, b2.reshape(1, D))


def kernel(x, edge_index, batch, W1, b1, W2, b2):
    src = edge_index[0]
    dst = edge_index[1]
    pad = EPAD - E
    srcp = jnp.concatenate(
        [src, jnp.zeros((pad,), jnp.int32)]).reshape(NS, NCH, CB)
    dstp = jnp.concatenate(
        [dst, jnp.full((pad,), JUNK, jnp.int32)]).reshape(NS, NCH, CB)
    zrows = jnp.zeros((NPAD, DH), jnp.float32)
    srcp, dstp, zrows = lax.optimization_barrier((srcp, dstp, zrows))

    deg_parts = _deg_call(dst)
    dinv = _dinv_call(deg_parts)
    dcol = dinv.reshape(NPAD, 1)[:N]

    xs2 = _scale_call(x, dcol)
    p2 = _agg_call(xs2, srcp, dstp, zrows)
    ys2 = _mid_call(xs2, p2[:, :N], dcol, W1, b1, W2)
    q2 = _agg_call(ys2, srcp, dstp, zrows)
    out = _fin_call(ys2, q2[:, :N], dcol, b2)
    return (out, None)
